# baseline (speedup 1.0000x reference)
import math

import jax
import jax.numpy as jnp
from jax import lax
from jax.experimental import pallas as pl
from jax.experimental.pallas import tpu as pltpu
from jax.experimental.pallas import tpu_sc as plsc

N_BATCH = 64
N_ATOMS = 64
N_G = 20
ETA = 2.3
RCUT = 6.0
KRAD = math.sqrt(ETA / math.pi)

NW = 32
MOL_PER_W = N_BATCH // NW

W_A = [1.0, 1.0, 1.0, 1.0, 1.0, 2.0, 1.0, 2.0, 2.0, 1.0]
A_OF_L = [[0], [1, 2, 3], [4, 5, 6, 7, 8, 9]]
COMBOS = [(0, 1), (0, 2), (0, 3), (1, 2), (1, 3), (2, 3)]
OUT_D = 3 * 10 * N_G


def _sc_body(coords, charges, out, cbuf, qbuf, xs, ys, zs, jord,
             prow, tbuf, orow_a, orow_b, sem_a, sem_b):
    cid = lax.axis_index("c")
    sid = lax.axis_index("s")
    wid = sid * 2 + cid
    lanes = lax.iota(jnp.int32, 16)
    lanes_f = lanes.astype(jnp.float32)
    off0 = (lanes_f + 1.0) * (RCUT / N_G)
    off1 = jnp.where(lanes < 4, (lanes_f + 17.0) * (RCUT / N_G), 1e9)
    zero16 = jnp.zeros((16,), jnp.float32)
    zidx = jnp.zeros((16,), jnp.int32)

    pltpu.sync_copy(coords, cbuf)
    pltpu.sync_copy(charges, qbuf)

    copies = []
    for m in range(MOL_PER_W):
        b = wid * MOL_PER_W + m
        orow = orow_a if m == 0 else orow_b
        sem = sem_a if m == 0 else sem_b
        ends = []
        cur = jnp.int32(0)
        for s_ch in (1, 6, 7, 8):
            for c4 in range(4):
                qv = qbuf[pl.ds(b * N_ATOMS + 16 * c4, 16)]
                ms = qv == s_ch
                pos = cur + plsc.cumsum(
                    jnp.where(ms, 1, 0).astype(jnp.int32)) - 1
                plsc.store_scatter(jord, [pos], lanes + 16 * c4, mask=ms)
                cur = cur + plsc.all_reduce_population_count(ms)[0]
            ends.append(cur)
        segs = [(jnp.int32(0), ends[0]), (ends[0], ends[1]),
                (ends[1], ends[2]), (ends[2], jnp.int32(N_ATOMS))]
        for c4 in range(4):
            sl = pl.ds(16 * c4, 16)
            jg = jord[sl]
            fl = b * (3 * N_ATOMS) + 3 * jg
            xs[sl] = plsc.load_gather(cbuf, [fl])
            ys[sl] = plsc.load_gather(cbuf, [fl + 1])
            zs[sl] = plsc.load_gather(cbuf, [fl + 2])

        def atom_body(i, carry):
            xi = xs[pl.ds(i, 16)][0]
            yi = ys[pl.ds(i, 16)][0]
            zi = zs[pl.ds(i, 16)][0]
            ri = jord[pl.ds(i, 16)][0]
            curs = [segs[s][0] for s in range(4)]
            for c4 in range(4):
                sl = pl.ds(16 * c4, 16)
                dx = xi - xs[sl]
                dy = yi - ys[sl]
                dz = zi - zs[sl]
                d2 = dx * dx + dy * dy + dz * dz
                jv = lanes + 16 * c4
                msk = (d2 < RCUT * RCUT) & (jv != i)
                d2s = jnp.where(msk, d2, 1.0)
                y = plsc.bitcast(0x5F3759DF - (plsc.bitcast(d2s, jnp.int32) >> 1),
                                 jnp.float32)
                for _ in range(3):
                    y = y * (1.5 - 0.5 * d2s * y * y)
                d = d2s * y
                v = d * (math.pi / 12.0)
                v2 = v * v
                s = v * (1.0 + v2 * (-1.0 / 6 + v2 * (1.0 / 120 + v2 * (
                    -1.0 / 5040 + v2 * (1.0 / 362880 + v2 * (-1.0 / 39916800))))))
                cf = jnp.where(msk, (1.0 - s * s) * KRAD, 0.0)
                inv2 = y * y
                inv3 = inv2 * y
                inv4 = inv2 * inv2
                cols = [d, cf, inv2, dx * inv3, dy * inv3, dz * inv3,
                        dx * dx * inv4, dx * dy * inv4, dy * dy * inv4,
                        dx * dz * inv4, dy * dz * inv4, dz * dz * inv4]
                sv = (jnp.where(jv >= ends[0], 1, 0)
                      + jnp.where(jv >= ends[1], 1, 0)
                      + jnp.where(jv >= ends[2], 1, 0))
                pos = zidx
                for s in range(4):
                    ms = msk & (sv == s)
                    ranks = plsc.cumsum(jnp.where(ms, 1, 0).astype(jnp.int32))
                    pos = jnp.where(ms, curs[s] + ranks - 1, pos)
                    curs[s] = curs[s] + plsc.all_reduce_population_count(ms)[0]
                for col, val in enumerate(cols):
                    plsc.store_scatter(prow, [pos, zidx + col], val, mask=msk)
            def j_body(j, acc):
                row = prow[j, :]
                dj = row[0]
                cj = row[1]
                t0 = dj - off0
                e0 = jnp.exp(t0 * t0 * (-ETA)) * cj
                t1 = dj - off1
                e1 = jnp.exp(t1 * t1 * (-ETA)) * cj
                new = []
                for a in range(10):
                    av = row[2 + a]
                    new.append(acc[2 * a] + e0 * av)
                    new.append(acc[2 * a + 1] + e1 * av)
                return tuple(new)
            for s in range(4):
                lo = segs[s][0]
                hi = curs[s]
                acc = plsc.parallel_loop(
                    lo, hi, 1, unroll=4,
                    carry=tuple(zero16 for _ in range(20)))(j_body)
                for a in range(10):
                    tbuf[pl.ds(s * 320 + a * 32, 16)] = acc[2 * a]
                    tbuf[pl.ds(s * 320 + a * 32 + 16, 16)] = acc[2 * a + 1]

            def tv(s, a, c):
                return tbuf[pl.ds(s * 320 + a * 32 + 16 * c, 16)]
            for l in range(3):
                accs = [[zero16] * 10, [zero16] * 10]
                for c in range(2):
                    for a in A_OF_L[l]:
                        t4 = [tv(s, a, c) for s in range(4)]
                        w = W_A[a]
                        for s in range(4):
                            accs[c][s] = accs[c][s] + w * (t4[s] * t4[s])
                        for fc, (s1, s2) in enumerate(COMBOS):
                            accs[c][4 + fc] = accs[c][4 + fc] + (2.0 * w) * (t4[s1] * t4[s2])
                for f in range(10):
                    base = ri * OUT_D + l * 200 + f * 20
                    orow[pl.ds(base, 16)] = accs[0][f]
                    if l == 2 and f == 9:
                        plsc.store_scatter(orow, [base + 16 + lanes], accs[1][f],
                                           mask=lanes < 4)
                    else:
                        orow[pl.ds(base + 16, 16)] = accs[1][f]
            return carry

        lax.fori_loop(0, N_ATOMS, atom_body, 0)
        copies.append(pltpu.async_copy(
            orow.at[pl.ds(0, N_ATOMS * OUT_D)],
            out.at[pl.ds(b * (N_ATOMS * OUT_D), N_ATOMS * OUT_D)], sem))
    for cp in copies:
        cp.wait()


def _make_kernel():
    mesh = plsc.VectorSubcoreMesh(core_axis_name="c", subcore_axis_name="s")
    return pl.kernel(
        _sc_body,
        out_type=jax.ShapeDtypeStruct((N_BATCH * N_ATOMS * OUT_D,), jnp.float32),
        mesh=mesh,
        scratch_types=[
            pltpu.VMEM((N_BATCH * N_ATOMS * 3,), jnp.float32),
            pltpu.VMEM((N_BATCH * N_ATOMS,), jnp.int32),
            pltpu.VMEM((N_ATOMS + 16,), jnp.float32),
            pltpu.VMEM((N_ATOMS + 16,), jnp.float32),
            pltpu.VMEM((N_ATOMS + 16,), jnp.float32),
            pltpu.VMEM((N_ATOMS + 16,), jnp.int32),
            pltpu.VMEM((N_ATOMS, 16), jnp.float32),
            pltpu.VMEM((4 * 10 * 32,), jnp.float32),
            pltpu.VMEM((N_ATOMS * OUT_D + 16,), jnp.float32),
            pltpu.VMEM((N_ATOMS * OUT_D + 16,), jnp.float32),
            pltpu.SemaphoreType.DMA,
            pltpu.SemaphoreType.DMA,
        ],
        compiler_params=pltpu.CompilerParams(needs_layout_passes=False),
    )


_sc_kernel_cache = []


def kernel(coordinates, nuclear_charges):
    if not _sc_kernel_cache:
        _sc_kernel_cache.append(_make_kernel())
    flat = _sc_kernel_cache[0](coordinates.reshape(-1), nuclear_charges.reshape(-1))
    return flat.reshape(N_BATCH, N_ATOMS, OUT_D)


# speedup vs baseline: 3.0991x; 3.0991x over previous
import math

import jax
import jax.numpy as jnp
from jax import lax
from jax.experimental import pallas as pl
from jax.experimental.pallas import tpu as pltpu
from jax.experimental.pallas import tpu_sc as plsc

N_BATCH = 64
N_ATOMS = 64
N_G = 20
ETA = 2.3
RCUT = 6.0
KRAD = math.sqrt(ETA / math.pi)

NW = 32
MOL_PER_W = N_BATCH // NW

W_A = [1.0, 1.0, 1.0, 1.0, 1.0, 2.0, 1.0, 2.0, 2.0, 1.0]
A_OF_L = [[0], [1, 2, 3], [4, 5, 6, 7, 8, 9]]
COMBOS = [(0, 1), (0, 2), (0, 3), (1, 2), (1, 3), (2, 3)]
OUT_D = 3 * 10 * N_G


def _sc_body(coords, charges, out, cbuf, qbuf, xs, ys, zs, jord,
             prow, tbuf, orow_a, orow_b, sem_a, sem_b):
    cid = lax.axis_index("c")
    sid = lax.axis_index("s")
    wid = sid * 2 + cid
    lanes = lax.iota(jnp.int32, 16)
    lanes_f = lanes.astype(jnp.float32)
    off0 = (lanes_f + 1.0) * (RCUT / N_G)
    off1 = jnp.where(lanes < 4, (lanes_f + 17.0) * (RCUT / N_G), 1e9)
    zero16 = jnp.zeros((16,), jnp.float32)
    zidx = jnp.zeros((16,), jnp.int32)

    pltpu.sync_copy(coords, cbuf)
    pltpu.sync_copy(charges, qbuf)

    copies = []
    for m in range(MOL_PER_W):
        b = wid * MOL_PER_W + m
        orow = orow_a if m == 0 else orow_b
        sem = sem_a if m == 0 else sem_b
        ends = []
        cur = jnp.int32(0)
        for s_ch in (1, 6, 7, 8):
            for c4 in range(4):
                qv = qbuf[pl.ds(b * N_ATOMS + 16 * c4, 16)]
                ms = qv == s_ch
                pos = cur + plsc.cumsum(
                    jnp.where(ms, 1, 0).astype(jnp.int32)) - 1
                plsc.store_scatter(jord, [pos], lanes + 16 * c4, mask=ms)
                cur = cur + plsc.all_reduce_population_count(ms)[0]
            ends.append(cur)
        segs = [(jnp.int32(0), ends[0]), (ends[0], ends[1]),
                (ends[1], ends[2]), (ends[2], jnp.int32(N_ATOMS))]
        for c4 in range(4):
            sl = pl.ds(16 * c4, 16)
            jg = jord[sl]
            fl = b * (3 * N_ATOMS) + 3 * jg
            xs[sl] = plsc.load_gather(cbuf, [fl])
            ys[sl] = plsc.load_gather(cbuf, [fl + 1])
            zs[sl] = plsc.load_gather(cbuf, [fl + 2])

        def atom_body(i, carry):
            xi = xs[pl.ds(i, 16)][0]
            yi = ys[pl.ds(i, 16)][0]
            zi = zs[pl.ds(i, 16)][0]
            ri = jord[pl.ds(i, 16)][0]
            curs = [segs[s][0] for s in range(4)]
            for c4 in range(4):
                sl = pl.ds(16 * c4, 16)
                dx = xi - xs[sl]
                dy = yi - ys[sl]
                dz = zi - zs[sl]
                d2 = dx * dx + dy * dy + dz * dz
                jv = lanes + 16 * c4
                msk = (d2 < RCUT * RCUT) & (jv != i)
                d2s = jnp.where(msk, d2, 1.0)
                y = plsc.bitcast(0x5F3759DF - (plsc.bitcast(d2s, jnp.int32) >> 1),
                                 jnp.float32)
                for _ in range(3):
                    y = y * (1.5 - 0.5 * d2s * y * y)
                d = d2s * y
                v = d * (math.pi / 12.0)
                v2 = v * v
                s = v * (1.0 + v2 * (-1.0 / 6 + v2 * (1.0 / 120 + v2 * (
                    -1.0 / 5040 + v2 * (1.0 / 362880 + v2 * (-1.0 / 39916800))))))
                cf = jnp.where(msk, (1.0 - s * s) * KRAD, 0.0)
                inv2 = y * y
                inv3 = inv2 * y
                inv4 = inv2 * inv2
                cols = [d, cf, inv2, dx * inv3, dy * inv3, dz * inv3,
                        dx * dx * inv4, dx * dy * inv4, dy * dy * inv4,
                        dx * dz * inv4, dy * dz * inv4, dz * dz * inv4]
                sv = (jnp.where(jv >= ends[0], 1, 0)
                      + jnp.where(jv >= ends[1], 1, 0)
                      + jnp.where(jv >= ends[2], 1, 0))
                pos = zidx
                for s in range(4):
                    ms = msk & (sv == s)
                    ranks = plsc.cumsum(jnp.where(ms, 1, 0).astype(jnp.int32))
                    pos = jnp.where(ms, curs[s] + ranks - 1, pos)
                    curs[s] = curs[s] + plsc.all_reduce_population_count(ms)[0]
                for col, val in enumerate(cols):
                    plsc.store_scatter(prow, [pos, zidx + col], val, mask=msk)
            def j_body(j, acc):
                row = prow[j, :]
                dj = row[0]
                cj = row[1]
                t0 = dj - off0
                e0 = jnp.exp(t0 * t0 * (-ETA)) * cj
                t1 = dj - off1
                e1 = jnp.exp(t1 * t1 * (-ETA)) * cj
                new = []
                for a in range(10):
                    av = row[2 + a]
                    new.append(acc[2 * a] + e0 * av)
                    new.append(acc[2 * a + 1] + e1 * av)
                return tuple(new)
            for s in range(4):
                lo = segs[s][0]
                hi = curs[s]
                acc = plsc.parallel_loop(
                    lo, hi, 1, unroll=2,
                    carry=tuple(zero16 for _ in range(20)))(j_body)
                for a in range(10):
                    tbuf[pl.ds(s * 320 + a * 32, 16)] = acc[2 * a]
                    tbuf[pl.ds(s * 320 + a * 32 + 16, 16)] = acc[2 * a + 1]

            def tv(s, a, c):
                return tbuf[pl.ds(s * 320 + a * 32 + 16 * c, 16)]
            for l in range(3):
                accs = [[zero16] * 10, [zero16] * 10]
                for c in range(2):
                    for a in A_OF_L[l]:
                        t4 = [tv(s, a, c) for s in range(4)]
                        w = W_A[a]
                        for s in range(4):
                            accs[c][s] = accs[c][s] + w * (t4[s] * t4[s])
                        for fc, (s1, s2) in enumerate(COMBOS):
                            accs[c][4 + fc] = accs[c][4 + fc] + (2.0 * w) * (t4[s1] * t4[s2])
                for f in range(10):
                    base = ri * OUT_D + l * 200 + f * 20
                    orow[pl.ds(base, 16)] = accs[0][f]
                    if l == 2 and f == 9:
                        plsc.store_scatter(orow, [base + 16 + lanes], accs[1][f],
                                           mask=lanes < 4)
                    else:
                        orow[pl.ds(base + 16, 16)] = accs[1][f]
            return carry

        lax.fori_loop(0, N_ATOMS, atom_body, 0)
        copies.append(pltpu.async_copy(
            orow.at[pl.ds(0, N_ATOMS * OUT_D)],
            out.at[pl.ds(b * (N_ATOMS * OUT_D), N_ATOMS * OUT_D)], sem))
    for cp in copies:
        cp.wait()


def _make_kernel():
    mesh = plsc.VectorSubcoreMesh(core_axis_name="c", subcore_axis_name="s")
    return pl.kernel(
        _sc_body,
        out_type=jax.ShapeDtypeStruct((N_BATCH * N_ATOMS * OUT_D,), jnp.float32),
        mesh=mesh,
        scratch_types=[
            pltpu.VMEM((N_BATCH * N_ATOMS * 3,), jnp.float32),
            pltpu.VMEM((N_BATCH * N_ATOMS,), jnp.int32),
            pltpu.VMEM((N_ATOMS + 16,), jnp.float32),
            pltpu.VMEM((N_ATOMS + 16,), jnp.float32),
            pltpu.VMEM((N_ATOMS + 16,), jnp.float32),
            pltpu.VMEM((N_ATOMS + 16,), jnp.int32),
            pltpu.VMEM((N_ATOMS, 16), jnp.float32),
            pltpu.VMEM((4 * 10 * 32,), jnp.float32),
            pltpu.VMEM((N_ATOMS * OUT_D + 16,), jnp.float32),
            pltpu.VMEM((N_ATOMS * OUT_D + 16,), jnp.float32),
            pltpu.SemaphoreType.DMA,
            pltpu.SemaphoreType.DMA,
        ],
        compiler_params=pltpu.CompilerParams(needs_layout_passes=False),
    )


_sc_kernel_cache = []


def kernel(coordinates, nuclear_charges):
    if not _sc_kernel_cache:
        _sc_kernel_cache.append(_make_kernel())
    flat = _sc_kernel_cache[0](coordinates.reshape(-1), nuclear_charges.reshape(-1))
    return flat.reshape(N_BATCH, N_ATOMS, OUT_D)


# speedup vs baseline: 3.1051x; 1.0019x over previous
import math

import jax
import jax.numpy as jnp
from jax import lax
from jax.experimental import pallas as pl
from jax.experimental.pallas import tpu as pltpu
from jax.experimental.pallas import tpu_sc as plsc

N_BATCH = 64
N_ATOMS = 64
N_G = 20
ETA = 2.3
RCUT = 6.0
KRAD = math.sqrt(ETA / math.pi)

NW = 32
MOL_PER_W = N_BATCH // NW

W_A = [1.0, 1.0, 1.0, 1.0, 1.0, 2.0, 1.0, 2.0, 2.0, 1.0]
A_OF_L = [[0], [1, 2, 3], [4, 5, 6, 7, 8, 9]]
COMBOS = [(0, 1), (0, 2), (0, 3), (1, 2), (1, 3), (2, 3)]
OUT_D = 3 * 10 * N_G


def _sc_body(coords, charges, out, cbuf, qbuf, xs, ys, zs, jord, sseg,
             prow, tbuf, orow_a, orow_b, sem_a, sem_b):
    cid = lax.axis_index("c")
    sid = lax.axis_index("s")
    wid = sid * 2 + cid
    lanes = lax.iota(jnp.int32, 16)
    lanes_f = lanes.astype(jnp.float32)
    off0 = (lanes_f + 1.0) * (RCUT / N_G)
    off1 = jnp.where(lanes < 4, (lanes_f + 17.0) * (RCUT / N_G), 1e9)
    zero16 = jnp.zeros((16,), jnp.float32)
    zidx = jnp.zeros((16,), jnp.int32)

    pltpu.sync_copy(coords, cbuf)
    pltpu.sync_copy(charges, qbuf)

    copies = []
    for m in range(MOL_PER_W):
        b = wid * MOL_PER_W + m
        orow = orow_a if m == 0 else orow_b
        sem = sem_a if m == 0 else sem_b
        ends = []
        cur = jnp.int32(0)
        for s_ch in (1, 6, 7, 8):
            for c4 in range(4):
                qv = qbuf[pl.ds(b * N_ATOMS + 16 * c4, 16)]
                ms = qv == s_ch
                pos = cur + plsc.cumsum(
                    jnp.where(ms, 1, 0).astype(jnp.int32)) - 1
                plsc.store_scatter(jord, [pos], lanes + 16 * c4, mask=ms)
                cur = cur + plsc.all_reduce_population_count(ms)[0]
            ends.append(cur)
        segs = [(jnp.int32(0), ends[0]), (ends[0], ends[1]),
                (ends[1], ends[2]), (ends[2], jnp.int32(N_ATOMS))]
        for c4 in range(4):
            sl = pl.ds(16 * c4, 16)
            jg = jord[sl]
            fl = b * (3 * N_ATOMS) + 3 * jg
            xs[sl] = plsc.load_gather(cbuf, [fl])
            ys[sl] = plsc.load_gather(cbuf, [fl + 1])
            zs[sl] = plsc.load_gather(cbuf, [fl + 2])
            jv = lanes + 16 * c4
            sseg[sl] = (jnp.where(jv >= ends[0], 1, 0)
                        + jnp.where(jv >= ends[1], 1, 0)
                        + jnp.where(jv >= ends[2], 1, 0))

        def atom_body(i, carry):
            xi = xs[pl.ds(i, 16)][0]
            yi = ys[pl.ds(i, 16)][0]
            zi = zs[pl.ds(i, 16)][0]
            ri = jord[pl.ds(i, 16)][0]
            curs = [segs[s][0] for s in range(4)]
            for c4 in range(4):
                sl = pl.ds(16 * c4, 16)
                dx = xi - xs[sl]
                dy = yi - ys[sl]
                dz = zi - zs[sl]
                d2 = dx * dx + dy * dy + dz * dz
                jv = lanes + 16 * c4
                msk = (d2 < RCUT * RCUT) & (jv != i)
                d2s = jnp.where(msk, d2, 1.0)
                y = plsc.bitcast(0x5F3759DF - (plsc.bitcast(d2s, jnp.int32) >> 1),
                                 jnp.float32)
                for _ in range(3):
                    y = y * (1.5 - 0.5 * d2s * y * y)
                d = d2s * y
                v = d * (math.pi / 12.0)
                v2 = v * v
                s = v * (1.0 + v2 * (-1.0 / 6 + v2 * (1.0 / 120 + v2 * (
                    -1.0 / 5040 + v2 * (1.0 / 362880 + v2 * (-1.0 / 39916800))))))
                cf = jnp.where(msk, (1.0 - s * s) * KRAD, 0.0)
                inv2 = y * y
                inv3 = inv2 * y
                inv4 = inv2 * inv2
                cols = [d, cf, inv2, dx * inv3, dy * inv3, dz * inv3,
                        dx * dx * inv4, dx * dy * inv4, dy * dy * inv4,
                        dx * dz * inv4, dy * dz * inv4, dz * dz * inv4]
                sv = sseg[sl]
                tot = plsc.cumsum(jnp.where(msk, 1, 0).astype(jnp.int32))
                cnt = [plsc.all_reduce_population_count(msk & (sv == s))[0]
                       for s in range(4)]
                before = [None] * 4
                before[0] = jnp.int32(0)
                for s in range(1, 4):
                    before[s] = before[s - 1] + cnt[s - 1]
                offv = jnp.where(sv == 0, curs[0] - before[0],
                        jnp.where(sv == 1, curs[1] - before[1],
                         jnp.where(sv == 2, curs[2] - before[2],
                                   curs[3] - before[3])))
                pos = offv + tot - 1
                for s in range(4):
                    curs[s] = curs[s] + cnt[s]
                for col, val in enumerate(cols):
                    plsc.store_scatter(prow, [pos, zidx + col], val, mask=msk)
            def j_body(j, acc):
                row = prow[j, :]
                dj = row[0]
                cj = row[1]
                t0 = dj - off0
                e0 = jnp.exp(t0 * t0 * (-ETA)) * cj
                t1 = dj - off1
                e1 = jnp.exp(t1 * t1 * (-ETA)) * cj
                new = []
                for a in range(10):
                    av = row[2 + a]
                    new.append(acc[2 * a] + e0 * av)
                    new.append(acc[2 * a + 1] + e1 * av)
                return tuple(new)
            for s in range(4):
                lo = segs[s][0]
                hi = curs[s]
                acc = plsc.parallel_loop(
                    lo, hi, 1, unroll=2,
                    carry=tuple(zero16 for _ in range(20)))(j_body)
                for a in range(10):
                    tbuf[pl.ds(s * 320 + a * 32, 16)] = acc[2 * a]
                    tbuf[pl.ds(s * 320 + a * 32 + 16, 16)] = acc[2 * a + 1]

            def tv(s, a, c):
                return tbuf[pl.ds(s * 320 + a * 32 + 16 * c, 16)]
            for l in range(3):
                accs = [[zero16] * 10, [zero16] * 10]
                for c in range(2):
                    for a in A_OF_L[l]:
                        t4 = [tv(s, a, c) for s in range(4)]
                        w = W_A[a]
                        for s in range(4):
                            accs[c][s] = accs[c][s] + w * (t4[s] * t4[s])
                        for fc, (s1, s2) in enumerate(COMBOS):
                            accs[c][4 + fc] = accs[c][4 + fc] + (2.0 * w) * (t4[s1] * t4[s2])
                for f in range(10):
                    base = ri * OUT_D + l * 200 + f * 20
                    orow[pl.ds(base, 16)] = accs[0][f]
                    if l == 2 and f == 9:
                        plsc.store_scatter(orow, [base + 16 + lanes], accs[1][f],
                                           mask=lanes < 4)
                    else:
                        orow[pl.ds(base + 16, 16)] = accs[1][f]
            return carry

        lax.fori_loop(0, N_ATOMS, atom_body, 0)
        copies.append(pltpu.async_copy(
            orow.at[pl.ds(0, N_ATOMS * OUT_D)],
            out.at[pl.ds(b * (N_ATOMS * OUT_D), N_ATOMS * OUT_D)], sem))
    for cp in copies:
        cp.wait()


def _make_kernel():
    mesh = plsc.VectorSubcoreMesh(core_axis_name="c", subcore_axis_name="s")
    return pl.kernel(
        _sc_body,
        out_type=jax.ShapeDtypeStruct((N_BATCH * N_ATOMS * OUT_D,), jnp.float32),
        mesh=mesh,
        scratch_types=[
            pltpu.VMEM((N_BATCH * N_ATOMS * 3,), jnp.float32),
            pltpu.VMEM((N_BATCH * N_ATOMS,), jnp.int32),
            pltpu.VMEM((N_ATOMS + 16,), jnp.float32),
            pltpu.VMEM((N_ATOMS + 16,), jnp.float32),
            pltpu.VMEM((N_ATOMS + 16,), jnp.float32),
            pltpu.VMEM((N_ATOMS + 16,), jnp.int32),
            pltpu.VMEM((N_ATOMS,), jnp.int32),
            pltpu.VMEM((N_ATOMS, 16), jnp.float32),
            pltpu.VMEM((4 * 10 * 32,), jnp.float32),
            pltpu.VMEM((N_ATOMS * OUT_D + 16,), jnp.float32),
            pltpu.VMEM((N_ATOMS * OUT_D + 16,), jnp.float32),
            pltpu.SemaphoreType.DMA,
            pltpu.SemaphoreType.DMA,
        ],
        compiler_params=pltpu.CompilerParams(needs_layout_passes=False),
    )


_sc_kernel_cache = []


def kernel(coordinates, nuclear_charges):
    if not _sc_kernel_cache:
        _sc_kernel_cache.append(_make_kernel())
    flat = _sc_kernel_cache[0](coordinates.reshape(-1), nuclear_charges.reshape(-1))
    return flat.reshape(N_BATCH, N_ATOMS, OUT_D)
